# baseline (speedup 1.0000x reference)
import jax
import jax.numpy as jnp
from jax.experimental import pallas as pl

_EARTH_R = 6371.0 * 1000.0
_N_OBS = 20000
_OBS_PAD = 20480
_BLK = 1024
_KSEL = 64
_K = 50


def _dot_block_kernel(qx_ref, qy_ref, qz_ref, ox_ref, oy_ref, oz_ref, out_ref):
    i = pl.program_id(0)
    dot = (qx_ref[...] * ox_ref[...]
           + qy_ref[...] * oy_ref[...]
           + qz_ref[...] * oz_ref[...])
    col = jax.lax.broadcasted_iota(jnp.int32, dot.shape, 1) + i * _BLK
    out_ref[...] = jnp.where(col < _N_OBS, dot, -3.0)


def _dot_matrix(qx, qy, qz, ox, oy, oz):
    n_q = qx.shape[0]
    grid = (_OBS_PAD // _BLK,)
    qspec = pl.BlockSpec((n_q, 1), lambda i: (0, 0))
    ospec = pl.BlockSpec((1, _BLK), lambda i: (0, i))
    return pl.pallas_call(
        _dot_block_kernel,
        grid=grid,
        in_specs=[qspec, qspec, qspec, ospec, ospec, ospec],
        out_specs=pl.BlockSpec((n_q, _BLK), lambda i: (0, i)),
        out_shape=jax.ShapeDtypeStruct((n_q, _OBS_PAD), jnp.float32),
    )(qx[:, None], qy[:, None], qz[:, None],
      ox[None, :], oy[None, :], oz[None, :])


def _exact_haversine(lat1, lng1, lat2, lng2):
    dlat = lat2 - lat1
    dlng = lng2 - lng1
    a = jnp.sin(dlat / 2.0) ** 2 + jnp.cos(lat1) * jnp.cos(lat2) * jnp.sin(dlng / 2.0) ** 2
    a = jnp.clip(a, 1e-12, 1.0 - 1e-9)
    return 2.0 * jnp.arcsin(jnp.sqrt(a))


def kernel(X, obs_idx, lat, lng):
    q_lat = jnp.take(lat, X)
    q_lng = jnp.take(lng, X)
    o_lat = jnp.take(lat, obs_idx)
    o_lng = jnp.take(lng, obs_idx)

    def to3d(la, ln):
        cl = jnp.cos(la)
        return cl * jnp.cos(ln), cl * jnp.sin(ln), jnp.sin(la)

    qx, qy, qz = to3d(q_lat, q_lng)
    ox, oy, oz = to3d(o_lat, o_lng)
    pad = _OBS_PAD - _N_OBS
    ox = jnp.pad(ox, (0, pad))
    oy = jnp.pad(oy, (0, pad))
    oz = jnp.pad(oz, (0, pad))

    dots = _dot_matrix(qx, qy, qz, ox, oy, oz)

    _, cand = jax.lax.top_k(dots, _KSEL)
    c_lat = jnp.take(o_lat, cand)
    c_lng = jnp.take(o_lng, cand)
    d = _exact_haversine(q_lat[:, None], q_lng[:, None], c_lat, c_lng)
    d_s, i_s = jax.lax.sort((d, cand), dimension=-1, num_keys=2)
    return d_s[:, :_K] * _EARTH_R, i_s[:, :_K]


# speedup vs baseline: 1.9143x; 1.9143x over previous
import jax
import jax.numpy as jnp
from jax.experimental import pallas as pl

_EARTH_R = 6371.0 * 1000.0
_N_OBS = 20000
_OBS_PAD = 20480
_BLKO = 256
_NSUB = _OBS_PAD // 8
_KSEL = 64
_K = 50
_IMIN = jnp.iinfo(jnp.int32).min


def _keys_kernel(ox_ref, oy_ref, oz_ref, qx_ref, qy_ref, qz_ref,
                 k1_ref, k2_ref, k3_ref):
    i = pl.program_id(0)
    dot = (ox_ref[...] * qx_ref[...]
           + oy_ref[...] * qy_ref[...]
           + oz_ref[...] * qz_ref[...])
    pos = jax.lax.broadcasted_iota(jnp.int32, dot.shape, 0) + i * _BLKO
    dot = jnp.where(pos < _N_OBS, dot, -3.0)
    key = jax.lax.bitcast_convert_type(dot + 2.0, jnp.int32)
    key = (key & ~7) | (7 - (pos & 7))
    kk = key.reshape(_BLKO // 8, 8, dot.shape[1])
    m1 = jnp.max(kk, axis=1)
    kk = jnp.where(kk == m1[:, None, :], _IMIN, kk)
    m2 = jnp.max(kk, axis=1)
    kk = jnp.where(kk == m2[:, None, :], _IMIN, kk)
    m3 = jnp.max(kk, axis=1)
    k1_ref[...] = m1
    k2_ref[...] = m2
    k3_ref[...] = m3


def _subgroup_keys(ox, oy, oz, qx, qy, qz):
    n_q = qx.shape[0]
    grid = (_OBS_PAD // _BLKO,)
    ospec = pl.BlockSpec((_BLKO, 1), lambda i: (i, 0))
    qspec = pl.BlockSpec((1, n_q), lambda i: (0, 0))
    kspec = pl.BlockSpec((_BLKO // 8, n_q), lambda i: (i, 0))
    return pl.pallas_call(
        _keys_kernel,
        grid=grid,
        in_specs=[ospec, ospec, ospec, qspec, qspec, qspec],
        out_specs=[kspec, kspec, kspec],
        out_shape=[jax.ShapeDtypeStruct((_NSUB, n_q), jnp.int32)] * 3,
    )(ox[:, None], oy[:, None], oz[:, None],
      qx[None, :], qy[None, :], qz[None, :])


def _exact_haversine(lat1, lng1, lat2, lng2):
    dlat = lat2 - lat1
    dlng = lng2 - lng1
    a = jnp.sin(dlat / 2.0) ** 2 + jnp.cos(lat1) * jnp.cos(lat2) * jnp.sin(dlng / 2.0) ** 2
    a = jnp.clip(a, 1e-12, 1.0 - 1e-9)
    return 2.0 * jnp.arcsin(jnp.sqrt(a))


def kernel(X, obs_idx, lat, lng):
    q_lat = jnp.take(lat, X)
    q_lng = jnp.take(lng, X)
    o_lat = jnp.take(lat, obs_idx)
    o_lng = jnp.take(lng, obs_idx)

    def to3d(la, ln):
        cl = jnp.cos(la)
        return cl * jnp.cos(ln), cl * jnp.sin(ln), jnp.sin(la)

    qx, qy, qz = to3d(q_lat, q_lng)
    ox, oy, oz = to3d(o_lat, o_lng)
    pad = _OBS_PAD - _N_OBS
    ox = jnp.pad(ox, (0, pad))
    oy = jnp.pad(oy, (0, pad))
    oz = jnp.pad(oz, (0, pad))

    k1, k2, k3 = _subgroup_keys(ox, oy, oz, qx, qy, qz)

    cat = jnp.concatenate([k1.T, k2.T, k3.T], axis=1)
    v, c = jax.lax.top_k(cat, _KSEL)
    cand = (c % _NSUB) * 8 + (7 - (v & 7))

    c_lat = jnp.take(o_lat, cand)
    c_lng = jnp.take(o_lng, cand)
    d = _exact_haversine(q_lat[:, None], q_lng[:, None], c_lat, c_lng)
    d_s, i_s = jax.lax.sort((d, cand), dimension=-1, num_keys=2)
    return d_s[:, :_K] * _EARTH_R, i_s[:, :_K]
